# baseline (speedup 1.0000x reference)
import jax
import jax.numpy as jnp
from jax import lax
from jax.experimental import pallas as pl
from jax.experimental.pallas import tpu as pltpu
from jax.experimental.pallas import tpu_sc as plsc

N = 10000
D_IN = 256
HEADS = 4
D_OUT = 64
HID = 128
E = 160000
HD = HEADS * D_OUT
HALF = HD // 2

BLK = 400
NBLK = N // BLK
NTILE = 16
EPT = E // NTILE
ECH = 80
NCH_E = EPT // ECH
NCH = 80
NCHTOT = N // NCH
NSLOT = -(-NCHTOT // NTILE)


def _s1_body(x_ref, w_ref, al_ref, ar_ref, feat_ref, elt_ref, ert_ref, m_ref, mx_ref):
    i = pl.program_id(0)
    feat = jnp.dot(x_ref[...], w_ref[...], preferred_element_type=jnp.float32)
    els, ers = [], []
    for h in range(HEADS):
        fh = feat[:, h * D_OUT:(h + 1) * D_OUT]
        els.append((fh * al_ref[h, :][None, :]).sum(axis=1))
        ers.append((fh * ar_ref[h, :][None, :]).sum(axis=1))
    el = jnp.stack(els, axis=1)
    er = jnp.stack(ers, axis=1)
    feat_ref[0, :, :] = feat[:, :HALF]
    feat_ref[1, :, :] = feat[:, HALF:]
    elt_ref[...] = el
    ert_ref[...] = er
    pad = jnp.full((12,), -1e30, jnp.float32)
    mrow = jnp.stack([jnp.concatenate([jnp.max(el, axis=0), pad]),
                      jnp.concatenate([jnp.max(er, axis=0), pad])], axis=0)

    @pl.when(i == 0)
    def _():
        m_ref[...] = mrow

    @pl.when(i != 0)
    def _():
        m_ref[...] = jnp.maximum(m_ref[...], mrow)

    @pl.when(i == NBLK - 1)
    def _():
        m = m_ref[...]
        mv = jnp.maximum(m[0:1, :] + m[1:2, :], 0.0)
        li = lax.broadcasted_iota(jnp.int32, (2, 16), 1) & 1
        cc = lax.broadcasted_iota(jnp.int32, (2, 16), 0)
        hsel = 2 * cc + li
        mx = jnp.zeros((2, 16), jnp.float32)
        for h in range(HEADS):
            mx = jnp.where(hsel == h, mv[:, h:h + 1], mx)
        mx_ref[...] = mx


def _stage1(x, W_gat, attn_l, attn_r):
    return pl.pallas_call(
        _s1_body,
        grid=(NBLK,),
        in_specs=[
            pl.BlockSpec((BLK, D_IN), lambda i: (i, 0)),
            pl.BlockSpec((D_IN, HD), lambda i: (0, 0)),
            pl.BlockSpec((HEADS, D_OUT), lambda i: (0, 0)),
            pl.BlockSpec((HEADS, D_OUT), lambda i: (0, 0)),
        ],
        out_specs=[
            pl.BlockSpec((2, BLK, HALF), lambda i: (0, i, 0)),
            pl.BlockSpec((BLK, HEADS), lambda i: (i, 0)),
            pl.BlockSpec((BLK, HEADS), lambda i: (i, 0)),
            pl.BlockSpec((2, 16), lambda i: (0, 0)),
            pl.BlockSpec((2, 16), lambda i: (0, 0)),
        ],
        out_shape=[
            jax.ShapeDtypeStruct((2, N, HALF), jnp.float32),
            jax.ShapeDtypeStruct((N, HEADS), jnp.float32),
            jax.ShapeDtypeStruct((N, HEADS), jnp.float32),
            jax.ShapeDtypeStruct((2, 16), jnp.float32),
            jax.ShapeDtypeStruct((2, 16), jnp.float32),
        ],
    )(x, W_gat, attn_l, attn_r)


def _bcast_lane(v, lane):
    idx = jnp.full((16, 1), lane, jnp.int32)
    return lax.gather(
        v, idx,
        lax.GatherDimensionNumbers(offset_dims=(), collapsed_slice_dims=(0,),
                                   start_index_map=(0,)),
        (1,), mode=lax.GatherScatterMode.PROMISE_IN_BOUNDS)


def _sc_body(feat2, elf, erf, mm, b2, zrows, src1, dst1, src2, dst2,
             h1o, h2o,
             gbufs, idss, ifas, ifbs, ifcs, ifds, iffs,
             g0s, g1s, g2s, g3s, lsrcs, ldsts,
             dbufA, dbufB, zbuf, bbuf, mtmp, sems, ssems, lsems,
             rst_sh, den0_sh, den1_sh):
    c = lax.axis_index("c")
    s = lax.axis_index("s")

    for q in range(ECH // 16):
        zbuf[pl.ds(q * 16, 16)] = jnp.zeros((16,), jnp.float32)

    pltpu.sync_copy(mm.at[c], mtmp)
    mvec = mtmp[...]
    m0 = mvec[0]
    m1 = mvec[1]
    pltpu.sync_copy(b2.at[c], bbuf)

    ebase = s * EPT
    h0 = 2 * c
    h1 = 2 * c + 1
    cn = c * N

    sets = tuple(
        (idss[i], ifas[i], ifbs[i], ifcs[i], ifds[i], iffs[i],
         g0s[i], g1s[i], g2s[i], g3s[i], gbufs[i], sems[i], ssems[i])
        for i in range(4))

    for (srcr, dstr, outr) in ((src1, dst1, h1o), (src2, dst2, h2o)):
        def _zero(k, _):
            ci = s + NTILE * k

            @pl.when(ci < NCHTOT)
            def _():
                n0 = pl.multiple_of(ci * NCH, NCH)
                pltpu.async_copy(zrows, rst_sh.at[pl.ds(n0, NCH)], sems[0])
                pltpu.async_copy(zbuf, den0_sh.at[pl.ds(n0, NCH)], sems[1])
                pltpu.async_copy(zbuf, den1_sh.at[pl.ds(n0, NCH)], sems[1])
            return 0
        lax.fori_loop(0, NSLOT, _zero, 0)

        def _zdrain(k, _):
            ci = s + NTILE * k

            @pl.when(ci < NCHTOT)
            def _():
                n0 = pl.multiple_of(ci * NCH, NCH)
                pltpu.make_async_copy(
                    zrows, rst_sh.at[pl.ds(n0, NCH)], sems[0]).wait()
                pltpu.make_async_copy(
                    zbuf, den0_sh.at[pl.ds(n0, NCH)], sems[1]).wait()
                pltpu.make_async_copy(
                    zbuf, den1_sh.at[pl.ds(n0, NCH)], sems[1]).wait()
            return 0
        lax.fori_loop(0, NSLOT, _zdrain, 0)
        plsc.subcore_barrier()

        def _lfire(ch, par):
            e0 = ebase + ch * ECH
            pltpu.async_copy(srcr.at[pl.ds(e0, ECH)], lsrcs[par], lsems[par])
            pltpu.async_copy(dstr.at[pl.ds(e0, ECH)], ldsts[par], lsems[par])

        def _prep(bs, ch, par, drain):
            ids, fa, fb, fc, fd, ff, g0, g1, g2, g3, gb, sem, ssem = bs
            if drain:
                @pl.when(ch >= 4)
                def _():
                    pltpu.make_async_copy(gb, rst_sh.at[ids], ssem).wait()
                    pltpu.make_async_copy(g0, den0_sh.at[ids], ssem).wait()
                    pltpu.make_async_copy(g1, den1_sh.at[ids], ssem).wait()
            e0 = ebase + ch * ECH
            lsr = lsrcs[par]
            lds = ldsts[par]
            pltpu.make_async_copy(
                srcr.at[pl.ds(e0, ECH)], lsr, lsems[par]).wait()
            pltpu.make_async_copy(
                dstr.at[pl.ds(e0, ECH)], lds, lsems[par]).wait()
            def _pidx(q, _):
                sl = pl.ds(q * 16, 16)
                sv = lsr[sl]
                dv = lds[sl]
                sv4 = sv * 4
                dv4 = dv * 4
                fa[sl] = sv4 + h0
                fb[sl] = sv4 + h1
                fc[sl] = dv4 + h0
                fd[sl] = dv4 + h1
                ff[sl] = sv + cn
                ids[sl] = dv
                return 0
            lax.fori_loop(0, ECH // 16, _pidx, 0)

            @pl.when(ch + 2 < NCH_E)
            def _():
                _lfire(ch + 2, par)
            pltpu.async_copy(elf.at[fa], g0, sem)
            pltpu.async_copy(elf.at[fb], g1, sem)
            pltpu.async_copy(erf.at[fc], g2, sem)
            pltpu.async_copy(erf.at[fd], g3, sem)
            pltpu.async_copy(feat2.at[ff], gb, sem)

        def _work(bs, ch):
            ids, fa, fb, fc, fd, ff, g0, g1, g2, g3, gb, sem, ssem = bs
            pltpu.make_async_copy(elf.at[fa], g0, sem).wait()
            pltpu.make_async_copy(elf.at[fb], g1, sem).wait()
            pltpu.make_async_copy(erf.at[fc], g2, sem).wait()
            pltpu.make_async_copy(erf.at[fd], g3, sem).wait()
            pltpu.make_async_copy(feat2.at[ff], gb, sem).wait()
            def _pee(q, _):
                sl = pl.ds(q * 16, 16)
                x0 = g0[sl] + g2[sl]
                x0 = jnp.maximum(x0, 0.2 * x0)
                v0 = jnp.exp(x0 - m0)
                x1 = g1[sl] + g3[sl]
                x1 = jnp.maximum(x1, 0.2 * x1)
                v1 = jnp.exp(x1 - m1)
                g0[sl] = v0
                g1[sl] = v1
                return 0
            lax.fori_loop(0, ECH // 16, _pee, 0)

            def _scale(e, _):
                base = (e >> 4) << 4
                lane = e & 15
                ea = g0[pl.ds(base, 16)]
                eb = g1[pl.ds(base, 16)]
                s0v = _bcast_lane(ea, lane)
                s1v = _bcast_lane(eb, lane)
                for j in range(8):
                    scv = s0v if j < 4 else s1v
                    gb[e, pl.ds(j * 16, 16)] = (
                        gb[e, pl.ds(j * 16, 16)] * scv)
                return 0
            lax.fori_loop(0, ECH, _scale, 0)
            pltpu.async_copy(gb, rst_sh.at[ids], ssem, add=True)
            pltpu.async_copy(g0, den0_sh.at[ids], ssem, add=True)
            pltpu.async_copy(g1, den1_sh.at[ids], ssem, add=True)

        _lfire(0, 0)
        _lfire(1, 1)
        _prep(sets[0], 0, 0, False)

        def _pmain(p, _):
            for b in range(4):
                ch = 4 * p + b

                @pl.when(ch + 1 < NCH_E)
                def _(ch=ch, b=b):
                    _prep(sets[(b + 1) % 4], ch + 1, (b + 1) % 2, True)

                @pl.when(ch < NCH_E)
                def _(ch=ch, b=b):
                    _work(sets[b], ch)
            return 0
        lax.fori_loop(0, (NCH_E + 3) // 4, _pmain, 0)
        for i in range(4):
            ids, fa, fb, fc, fd, ff, g0, g1, g2, g3, gb, sem, ssem = \
                sets[i]
            pltpu.make_async_copy(gb, rst_sh.at[ids], ssem).wait()
            pltpu.make_async_copy(g0, den0_sh.at[ids], ssem).wait()
            pltpu.make_async_copy(g1, den1_sh.at[ids], ssem).wait()

        plsc.subcore_barrier()

        def _p3prep(k, b):
            ci = s + NTILE * k

            @pl.when(ci < NCHTOT)
            def _():
                n0 = pl.multiple_of(ci * NCH, NCH)
                gb = gbufs[b]
                if True:
                    @pl.when(k >= 2)
                    def _():
                        ci2 = s + NTILE * (k - 2)
                        n2 = pl.multiple_of(ci2 * NCH, NCH)
                        pltpu.make_async_copy(
                            gb, outr.at[c, pl.ds(n2, NCH), :],
                            ssems[b]).wait()
                pltpu.async_copy(rst_sh.at[pl.ds(n0, NCH)], gb, sems[b])
                pltpu.async_copy(den0_sh.at[pl.ds(n0, NCH)],
                                 dbufA[b], sems[b])
                pltpu.async_copy(den1_sh.at[pl.ds(n0, NCH)],
                                 dbufB[b], sems[b])

        def _p3work(k, b):
            ci = s + NTILE * k

            @pl.when(ci < NCHTOT)
            def _():
                n0 = pl.multiple_of(ci * NCH, NCH)
                gb = gbufs[b]
                pltpu.make_async_copy(
                    rst_sh.at[pl.ds(n0, NCH)], gb, sems[b]).wait()
                pltpu.make_async_copy(
                    den0_sh.at[pl.ds(n0, NCH)], dbufA[b], sems[b]).wait()
                pltpu.make_async_copy(
                    den1_sh.at[pl.ds(n0, NCH)], dbufB[b], sems[b]).wait()

                def _p3(g, _):
                    r0v = 1.0 / jnp.maximum(dbufA[b][pl.ds(g * 16, 16)],
                                            1e-9)
                    r1v = 1.0 / jnp.maximum(dbufB[b][pl.ds(g * 16, 16)],
                                            1e-9)
                    for n16 in range(16):
                        n = g * 16 + n16
                        r0 = _bcast_lane(r0v, n16)
                        r1 = _bcast_lane(r1v, n16)
                        for j in range(8):
                            r = r0 if j < 4 else r1
                            v = (gb[n, pl.ds(j * 16, 16)] * r
                                 + bbuf[pl.ds(j * 16, 16)])
                            v = jnp.where(v > 0, v,
                                          jnp.exp(jnp.minimum(v, 0.0)) - 1.0)
                            gb[n, pl.ds(j * 16, 16)] = v
                    return 0
                lax.fori_loop(0, NCH // 16, _p3, 0)
                pltpu.async_copy(gb, outr.at[c, pl.ds(n0, NCH), :], ssems[b])

        _p3prep(0, 0)

        def _p3outer(q, _):
            for b in range(2):
                k = 2 * q + b
                if True:
                    @pl.when(k + 1 < NSLOT)
                    def _(k=k, b=b):
                        _p3prep(k + 1, 1 - b)
                _p3work(k, b)
            return 0
        lax.fori_loop(0, NSLOT // 2, _p3outer, 0)
        for k, b in ((NSLOT - 2, 0), (NSLOT - 1, 1)):
            ci = s + NTILE * k

            @pl.when(ci < NCHTOT)
            def _(k=k, b=b, ci=ci):
                n0 = pl.multiple_of(ci * NCH, NCH)
                pltpu.make_async_copy(
                    gbufs[b], outr.at[c, pl.ds(n0, NCH), :], ssems[b]).wait()
        plsc.subcore_barrier()


def _stage2(feat2, elf, erf, mm, b2, zrows, src1, dst1, src2, dst2):
    i32 = jnp.int32
    f32 = jnp.float32
    fn = pl.kernel(
        _sc_body,
        out_type=[jax.ShapeDtypeStruct((2, N, HALF), f32),
                  jax.ShapeDtypeStruct((2, N, HALF), f32)],
        mesh=plsc.VectorSubcoreMesh(core_axis_name="c", subcore_axis_name="s"),
        scratch_types=[
            [pltpu.VMEM((NCH, HALF), f32)] * 4,
            [pltpu.VMEM((ECH,), i32)] * 4,
            [pltpu.VMEM((ECH,), i32)] * 4,
            [pltpu.VMEM((ECH,), i32)] * 4,
            [pltpu.VMEM((ECH,), i32)] * 4,
            [pltpu.VMEM((ECH,), i32)] * 4,
            [pltpu.VMEM((ECH,), i32)] * 4,
            [pltpu.VMEM((ECH,), f32)] * 4,
            [pltpu.VMEM((ECH,), f32)] * 4,
            [pltpu.VMEM((ECH,), f32)] * 4,
            [pltpu.VMEM((ECH,), f32)] * 4,
            [pltpu.VMEM((ECH,), i32)] * 2,
            [pltpu.VMEM((ECH,), i32)] * 2,
            [pltpu.VMEM((NCH,), f32)] * 2,
            [pltpu.VMEM((NCH,), f32)] * 2,
            pltpu.VMEM((ECH,), f32),
            pltpu.VMEM((HALF,), f32),
            pltpu.VMEM((16,), f32),
            [pltpu.SemaphoreType.DMA] * 4,
            [pltpu.SemaphoreType.DMA] * 4,
            [pltpu.SemaphoreType.DMA] * 2,
            pltpu.VMEM_SHARED((N, HALF), f32),
            pltpu.VMEM_SHARED((N,), f32),
            pltpu.VMEM_SHARED((N,), f32),
        ],
    )
    return fn(feat2, elf, erf, mm, b2, zrows, src1, dst1, src2, dst2)


def _s3a_body(h1_ref, h2_ref, w1_ref, b1_ref, w2_ref, acc_ref):
    i = pl.program_id(0)
    z1 = jnp.concatenate([h1_ref[0], h1_ref[1]], axis=1)
    z2 = jnp.concatenate([h2_ref[0], h2_ref[1]], axis=1)
    t1 = jnp.tanh(jnp.dot(z1, w1_ref[...], preferred_element_type=jnp.float32)
                  + b1_ref[...])
    t2 = jnp.tanh(jnp.dot(z2, w1_ref[...], preferred_element_type=jnp.float32)
                  + b1_ref[...])
    s1 = jnp.sum(t1 * w2_ref[...])
    s2 = jnp.sum(t2 * w2_ref[...])
    row = jnp.stack([s1, s2]).reshape(1, 2)

    @pl.when(i == 0)
    def _():
        acc_ref[...] = row

    @pl.when(i != 0)
    def _():
        acc_ref[...] = acc_ref[...] + row


def _stage3a(h1h, h2h, W1, b1r, w2r):
    return pl.pallas_call(
        _s3a_body,
        grid=(NBLK,),
        in_specs=[
            pl.BlockSpec((2, BLK, HALF), lambda i: (0, i, 0)),
            pl.BlockSpec((2, BLK, HALF), lambda i: (0, i, 0)),
            pl.BlockSpec((HD, HID), lambda i: (0, 0)),
            pl.BlockSpec((1, HID), lambda i: (0, 0)),
            pl.BlockSpec((1, HID), lambda i: (0, 0)),
        ],
        out_specs=pl.BlockSpec((1, 2), lambda i: (0, 0)),
        out_shape=jax.ShapeDtypeStruct((1, 2), jnp.float32),
    )(h1h, h2h, W1, b1r, w2r)


def _s3b_body(acc_ref, h1_ref, h2_ref, out_ref):
    w0 = acc_ref[0, 0] / N
    w1 = acc_ref[0, 1] / N
    m = jnp.maximum(w0, w1)
    e0 = jnp.exp(w0 - m)
    e1 = jnp.exp(w1 - m)
    bb0 = e0 / (e0 + e1)
    bb1 = e1 / (e0 + e1)
    left = bb0 * h1_ref[0] + bb1 * h2_ref[0]
    right = bb0 * h1_ref[1] + bb1 * h2_ref[1]
    out_ref[...] = jnp.concatenate([left, right], axis=1)


def _stage3b(acc, h1h, h2h):
    return pl.pallas_call(
        _s3b_body,
        grid=(NBLK,),
        in_specs=[
            pl.BlockSpec((1, 2), lambda i: (0, 0)),
            pl.BlockSpec((2, BLK, HALF), lambda i: (0, i, 0)),
            pl.BlockSpec((2, BLK, HALF), lambda i: (0, i, 0)),
        ],
        out_specs=pl.BlockSpec((BLK, HD), lambda i: (i, 0)),
        out_shape=jax.ShapeDtypeStruct((N, HD), jnp.float32),
    )(acc, h1h, h2h)


def kernel(x, edge_index1, edge_index2, W_gat, attn_l, attn_r, b_gat,
           W1, b1, W2):
    src1, dst1 = edge_index1[0], edge_index1[1]
    src2, dst2 = edge_index2[0], edge_index2[1]
    feat_h, elt, ert, _mraw, mx = _stage1(x, W_gat, attn_l, attn_r)
    feat2 = feat_h.reshape(2 * N, HALF)
    b2 = b_gat.reshape(2, HALF)
    zrows = jnp.zeros((NCH, HALF), jnp.float32)
    elf = elt.reshape(HEADS * N)
    erf = ert.reshape(HEADS * N)
    h1h, h2h = _stage2(feat2, elf, erf, mx, b2, zrows, src1, dst1,
                       src2, dst2)
    acc = _stage3a(h1h, h2h, W1, b1.reshape(1, HID), W2.reshape(1, HID))
    return _stage3b(acc, h1h, h2h)


# speedup vs baseline: 1.0477x; 1.0477x over previous
import jax
import jax.numpy as jnp
from jax import lax
from jax.experimental import pallas as pl
from jax.experimental.pallas import tpu as pltpu
from jax.experimental.pallas import tpu_sc as plsc

N = 10000
D_IN = 256
HEADS = 4
D_OUT = 64
HID = 128
E = 160000
HD = HEADS * D_OUT
HALF = HD // 2

BLK = 400
NBLK = N // BLK
NTILE = 16
EPT = E // NTILE
ECH = 80
NCH_E = EPT // ECH
NCH = 80
NCHTOT = N // NCH
NSLOT = -(-NCHTOT // NTILE)


def _s1_body(x_ref, w_ref, al_ref, ar_ref, feat_ref, elt_ref, ert_ref, m_ref, mx_ref):
    i = pl.program_id(0)
    feat = jnp.dot(x_ref[...], w_ref[...], preferred_element_type=jnp.float32)
    els, ers = [], []
    for h in range(HEADS):
        fh = feat[:, h * D_OUT:(h + 1) * D_OUT]
        els.append((fh * al_ref[h, :][None, :]).sum(axis=1))
        ers.append((fh * ar_ref[h, :][None, :]).sum(axis=1))
    el = jnp.stack(els, axis=1)
    er = jnp.stack(ers, axis=1)
    feat_ref[0, :, :] = feat[:, :HALF]
    feat_ref[1, :, :] = feat[:, HALF:]
    elt_ref[...] = el
    ert_ref[...] = er
    pad = jnp.full((12,), -1e30, jnp.float32)
    mrow = jnp.stack([jnp.concatenate([jnp.max(el, axis=0), pad]),
                      jnp.concatenate([jnp.max(er, axis=0), pad])], axis=0)

    @pl.when(i == 0)
    def _():
        m_ref[...] = mrow

    @pl.when(i != 0)
    def _():
        m_ref[...] = jnp.maximum(m_ref[...], mrow)

    @pl.when(i == NBLK - 1)
    def _():
        m = m_ref[...]
        mv = jnp.maximum(m[0:1, :] + m[1:2, :], 0.0)
        li = lax.broadcasted_iota(jnp.int32, (2, 16), 1) & 1
        cc = lax.broadcasted_iota(jnp.int32, (2, 16), 0)
        hsel = 2 * cc + li
        mx = jnp.zeros((2, 16), jnp.float32)
        for h in range(HEADS):
            mx = jnp.where(hsel == h, mv[:, h:h + 1], mx)
        mx_ref[...] = mx


def _stage1(x, W_gat, attn_l, attn_r):
    return pl.pallas_call(
        _s1_body,
        grid=(NBLK,),
        in_specs=[
            pl.BlockSpec((BLK, D_IN), lambda i: (i, 0)),
            pl.BlockSpec((D_IN, HD), lambda i: (0, 0)),
            pl.BlockSpec((HEADS, D_OUT), lambda i: (0, 0)),
            pl.BlockSpec((HEADS, D_OUT), lambda i: (0, 0)),
        ],
        out_specs=[
            pl.BlockSpec((2, BLK, HALF), lambda i: (0, i, 0)),
            pl.BlockSpec((BLK, HEADS), lambda i: (i, 0)),
            pl.BlockSpec((BLK, HEADS), lambda i: (i, 0)),
            pl.BlockSpec((2, 16), lambda i: (0, 0)),
            pl.BlockSpec((2, 16), lambda i: (0, 0)),
        ],
        out_shape=[
            jax.ShapeDtypeStruct((2, N, HALF), jnp.float32),
            jax.ShapeDtypeStruct((N, HEADS), jnp.float32),
            jax.ShapeDtypeStruct((N, HEADS), jnp.float32),
            jax.ShapeDtypeStruct((2, 16), jnp.float32),
            jax.ShapeDtypeStruct((2, 16), jnp.float32),
        ],
    )(x, W_gat, attn_l, attn_r)


def _bcast_lane(v, lane):
    idx = jnp.full((16, 1), lane, jnp.int32)
    return lax.gather(
        v, idx,
        lax.GatherDimensionNumbers(offset_dims=(), collapsed_slice_dims=(0,),
                                   start_index_map=(0,)),
        (1,), mode=lax.GatherScatterMode.PROMISE_IN_BOUNDS)


def _sc_body(feat2, elf, erf, mm, b2, zrows, eis,
             hso,
             gbufs, idss, ifas, ifbs, ifcs, ifds, iffs,
             g0s, g1s, g2s, g3s, lsrcs, ldsts,
             dbufA, dbufB, zbuf, bbuf, mtmp, sems, ssems, lsems,
             rst_sh, den0_sh, den1_sh):
    c = lax.axis_index("c")
    s = lax.axis_index("s")

    for q in range(ECH // 16):
        zbuf[pl.ds(q * 16, 16)] = jnp.zeros((16,), jnp.float32)

    pltpu.sync_copy(mm.at[c], mtmp)
    mvec = mtmp[...]
    m0 = mvec[0]
    m1 = mvec[1]
    pltpu.sync_copy(b2.at[c], bbuf)

    ebase = s * EPT
    h0 = 2 * c
    h1 = 2 * c + 1
    cn = c * N

    sets = tuple(
        (idss[i], ifas[i], ifbs[i], ifcs[i], ifds[i], iffs[i],
         g0s[i], g1s[i], g2s[i], g3s[i], gbufs[i], sems[i], ssems[i])
        for i in range(4))

    def _meta(m, _carry):
        sbase = pl.multiple_of((2 * m) * E, 8)
        dbase = pl.multiple_of((2 * m + 1) * E, 8)
        def _zero(k, _):
            ci = s + NTILE * k

            @pl.when(ci < NCHTOT)
            def _():
                n0 = pl.multiple_of(ci * NCH, NCH)
                pltpu.async_copy(zrows, rst_sh.at[pl.ds(n0, NCH)], sems[0])
                pltpu.async_copy(zbuf, den0_sh.at[pl.ds(n0, NCH)], sems[1])
                pltpu.async_copy(zbuf, den1_sh.at[pl.ds(n0, NCH)], sems[1])
            return 0
        lax.fori_loop(0, NSLOT, _zero, 0)

        def _zdrain(k, _):
            ci = s + NTILE * k

            @pl.when(ci < NCHTOT)
            def _():
                n0 = pl.multiple_of(ci * NCH, NCH)
                pltpu.make_async_copy(
                    zrows, rst_sh.at[pl.ds(n0, NCH)], sems[0]).wait()
                pltpu.make_async_copy(
                    zbuf, den0_sh.at[pl.ds(n0, NCH)], sems[1]).wait()
                pltpu.make_async_copy(
                    zbuf, den1_sh.at[pl.ds(n0, NCH)], sems[1]).wait()
            return 0
        lax.fori_loop(0, NSLOT, _zdrain, 0)
        plsc.subcore_barrier()

        def _lfire(ch, par):
            e0 = ebase + ch * ECH
            pltpu.async_copy(eis.at[pl.ds(sbase + e0, ECH)], lsrcs[par],
                             lsems[par])
            pltpu.async_copy(eis.at[pl.ds(dbase + e0, ECH)], ldsts[par],
                             lsems[par])

        def _prep(bs, ch, par, drain):
            ids, fa, fb, fc, fd, ff, g0, g1, g2, g3, gb, sem, ssem = bs
            if drain:
                @pl.when(ch >= 4)
                def _():
                    pltpu.make_async_copy(gb, rst_sh.at[ids], ssem).wait()
                    pltpu.make_async_copy(g0, den0_sh.at[ids], ssem).wait()
                    pltpu.make_async_copy(g1, den1_sh.at[ids], ssem).wait()
            e0 = ebase + ch * ECH
            lsr = lsrcs[par]
            lds = ldsts[par]
            pltpu.make_async_copy(
                eis.at[pl.ds(sbase + e0, ECH)], lsr, lsems[par]).wait()
            pltpu.make_async_copy(
                eis.at[pl.ds(dbase + e0, ECH)], lds, lsems[par]).wait()
            def _pidx(q, _):
                sl = pl.ds(q * 16, 16)
                sv = lsr[sl]
                dv = lds[sl]
                sv4 = sv * 4
                dv4 = dv * 4
                fa[sl] = sv4 + h0
                fb[sl] = sv4 + h1
                fc[sl] = dv4 + h0
                fd[sl] = dv4 + h1
                ff[sl] = sv + cn
                ids[sl] = dv
                return 0
            lax.fori_loop(0, ECH // 16, _pidx, 0)

            @pl.when(ch + 2 < NCH_E)
            def _():
                _lfire(ch + 2, par)
            pltpu.async_copy(elf.at[fa], g0, sem)
            pltpu.async_copy(elf.at[fb], g1, sem)
            pltpu.async_copy(erf.at[fc], g2, sem)
            pltpu.async_copy(erf.at[fd], g3, sem)
            pltpu.async_copy(feat2.at[ff], gb, sem)

        def _work(bs, ch):
            ids, fa, fb, fc, fd, ff, g0, g1, g2, g3, gb, sem, ssem = bs
            pltpu.make_async_copy(elf.at[fa], g0, sem).wait()
            pltpu.make_async_copy(elf.at[fb], g1, sem).wait()
            pltpu.make_async_copy(erf.at[fc], g2, sem).wait()
            pltpu.make_async_copy(erf.at[fd], g3, sem).wait()
            pltpu.make_async_copy(feat2.at[ff], gb, sem).wait()
            def _pee(q, _):
                sl = pl.ds(q * 16, 16)
                x0 = g0[sl] + g2[sl]
                x0 = jnp.maximum(x0, 0.2 * x0)
                v0 = jnp.exp(x0 - m0)
                x1 = g1[sl] + g3[sl]
                x1 = jnp.maximum(x1, 0.2 * x1)
                v1 = jnp.exp(x1 - m1)
                g0[sl] = v0
                g1[sl] = v1
                return 0
            lax.fori_loop(0, ECH // 16, _pee, 0)

            def _scale(g, _):
                ea = g0[pl.ds(g * 16, 16)]
                eb = g1[pl.ds(g * 16, 16)]
                for e16 in range(16):
                    e = g * 16 + e16
                    s0v = _bcast_lane(ea, e16)
                    s1v = _bcast_lane(eb, e16)
                    for j in range(8):
                        scv = s0v if j < 4 else s1v
                        gb[e, pl.ds(j * 16, 16)] = (
                            gb[e, pl.ds(j * 16, 16)] * scv)
                return 0
            lax.fori_loop(0, ECH // 16, _scale, 0)
            pltpu.async_copy(gb, rst_sh.at[ids], ssem, add=True)
            pltpu.async_copy(g0, den0_sh.at[ids], ssem, add=True)
            pltpu.async_copy(g1, den1_sh.at[ids], ssem, add=True)

        _lfire(0, 0)
        _lfire(1, 1)
        _prep(sets[0], 0, 0, False)

        def _pmain(p, _):
            for b in range(4):
                ch = 4 * p + b

                @pl.when(ch + 1 < NCH_E)
                def _(ch=ch, b=b):
                    _prep(sets[(b + 1) % 4], ch + 1, (b + 1) % 2, True)

                @pl.when(ch < NCH_E)
                def _(ch=ch, b=b):
                    _work(sets[b], ch)
            return 0
        lax.fori_loop(0, (NCH_E + 3) // 4, _pmain, 0)
        for i in range(4):
            ids, fa, fb, fc, fd, ff, g0, g1, g2, g3, gb, sem, ssem = \
                sets[i]
            pltpu.make_async_copy(gb, rst_sh.at[ids], ssem).wait()
            pltpu.make_async_copy(g0, den0_sh.at[ids], ssem).wait()
            pltpu.make_async_copy(g1, den1_sh.at[ids], ssem).wait()

        plsc.subcore_barrier()

        def _p3prep(k, b):
            ci = s + NTILE * k

            @pl.when(ci < NCHTOT)
            def _():
                n0 = pl.multiple_of(ci * NCH, NCH)
                gb = gbufs[b]
                if True:
                    @pl.when(k >= 2)
                    def _():
                        ci2 = s + NTILE * (k - 2)
                        n2 = pl.multiple_of(ci2 * NCH, NCH)
                        pltpu.make_async_copy(
                            gb, hso.at[m, c, pl.ds(n2, NCH), :],
                            ssems[b]).wait()
                pltpu.async_copy(rst_sh.at[pl.ds(n0, NCH)], gb, sems[b])
                pltpu.async_copy(den0_sh.at[pl.ds(n0, NCH)],
                                 dbufA[b], sems[b])
                pltpu.async_copy(den1_sh.at[pl.ds(n0, NCH)],
                                 dbufB[b], sems[b])

        def _p3work(k, b):
            ci = s + NTILE * k

            @pl.when(ci < NCHTOT)
            def _():
                n0 = pl.multiple_of(ci * NCH, NCH)
                gb = gbufs[b]
                pltpu.make_async_copy(
                    rst_sh.at[pl.ds(n0, NCH)], gb, sems[b]).wait()
                pltpu.make_async_copy(
                    den0_sh.at[pl.ds(n0, NCH)], dbufA[b], sems[b]).wait()
                pltpu.make_async_copy(
                    den1_sh.at[pl.ds(n0, NCH)], dbufB[b], sems[b]).wait()

                def _p3(g, _):
                    r0v = 1.0 / jnp.maximum(dbufA[b][pl.ds(g * 16, 16)],
                                            1e-9)
                    r1v = 1.0 / jnp.maximum(dbufB[b][pl.ds(g * 16, 16)],
                                            1e-9)
                    for n16 in range(16):
                        n = g * 16 + n16
                        r0 = _bcast_lane(r0v, n16)
                        r1 = _bcast_lane(r1v, n16)
                        for j in range(8):
                            r = r0 if j < 4 else r1
                            v = (gb[n, pl.ds(j * 16, 16)] * r
                                 + bbuf[pl.ds(j * 16, 16)])
                            v = jnp.where(v > 0, v,
                                          jnp.exp(jnp.minimum(v, 0.0)) - 1.0)
                            gb[n, pl.ds(j * 16, 16)] = v
                    return 0
                lax.fori_loop(0, NCH // 16, _p3, 0)
                pltpu.async_copy(gb, hso.at[m, c, pl.ds(n0, NCH), :], ssems[b])

        _p3prep(0, 0)

        def _p3outer(q, _):
            for b in range(2):
                k = 2 * q + b
                if True:
                    @pl.when(k + 1 < NSLOT)
                    def _(k=k, b=b):
                        _p3prep(k + 1, 1 - b)
                _p3work(k, b)
            return 0
        lax.fori_loop(0, NSLOT // 2, _p3outer, 0)
        for k, b in ((NSLOT - 2, 0), (NSLOT - 1, 1)):
            ci = s + NTILE * k

            @pl.when(ci < NCHTOT)
            def _(k=k, b=b, ci=ci):
                n0 = pl.multiple_of(ci * NCH, NCH)
                pltpu.make_async_copy(
                    gbufs[b], hso.at[m, c, pl.ds(n0, NCH), :], ssems[b]).wait()
        plsc.subcore_barrier()
        return 0

    lax.fori_loop(0, 2, _meta, 0)


def _stage2(feat2, elf, erf, mm, b2, zrows, eis):
    i32 = jnp.int32
    f32 = jnp.float32
    fn = pl.kernel(
        _sc_body,
        out_type=jax.ShapeDtypeStruct((2, 2, N, HALF), f32),
        mesh=plsc.VectorSubcoreMesh(core_axis_name="c", subcore_axis_name="s"),
        scratch_types=[
            [pltpu.VMEM((NCH, HALF), f32)] * 4,
            [pltpu.VMEM((ECH,), i32)] * 4,
            [pltpu.VMEM((ECH,), i32)] * 4,
            [pltpu.VMEM((ECH,), i32)] * 4,
            [pltpu.VMEM((ECH,), i32)] * 4,
            [pltpu.VMEM((ECH,), i32)] * 4,
            [pltpu.VMEM((ECH,), i32)] * 4,
            [pltpu.VMEM((ECH,), f32)] * 4,
            [pltpu.VMEM((ECH,), f32)] * 4,
            [pltpu.VMEM((ECH,), f32)] * 4,
            [pltpu.VMEM((ECH,), f32)] * 4,
            [pltpu.VMEM((ECH,), i32)] * 2,
            [pltpu.VMEM((ECH,), i32)] * 2,
            [pltpu.VMEM((NCH,), f32)] * 2,
            [pltpu.VMEM((NCH,), f32)] * 2,
            pltpu.VMEM((ECH,), f32),
            pltpu.VMEM((HALF,), f32),
            pltpu.VMEM((16,), f32),
            [pltpu.SemaphoreType.DMA] * 4,
            [pltpu.SemaphoreType.DMA] * 4,
            [pltpu.SemaphoreType.DMA] * 2,
            pltpu.VMEM_SHARED((N, HALF), f32),
            pltpu.VMEM_SHARED((N,), f32),
            pltpu.VMEM_SHARED((N,), f32),
        ],
    )
    return fn(feat2, elf, erf, mm, b2, zrows, eis)


def _s3a_body(h1_ref, h2_ref, w1_ref, b1_ref, w2_ref, acc_ref):
    i = pl.program_id(0)
    z1 = jnp.concatenate([h1_ref[0], h1_ref[1]], axis=1)
    z2 = jnp.concatenate([h2_ref[0], h2_ref[1]], axis=1)
    t1 = jnp.tanh(jnp.dot(z1, w1_ref[...], preferred_element_type=jnp.float32)
                  + b1_ref[...])
    t2 = jnp.tanh(jnp.dot(z2, w1_ref[...], preferred_element_type=jnp.float32)
                  + b1_ref[...])
    s1 = jnp.sum(t1 * w2_ref[...])
    s2 = jnp.sum(t2 * w2_ref[...])
    row = jnp.stack([s1, s2]).reshape(1, 2)

    @pl.when(i == 0)
    def _():
        acc_ref[...] = row

    @pl.when(i != 0)
    def _():
        acc_ref[...] = acc_ref[...] + row


def _stage3a(h1h, h2h, W1, b1r, w2r):
    return pl.pallas_call(
        _s3a_body,
        grid=(NBLK,),
        in_specs=[
            pl.BlockSpec((2, BLK, HALF), lambda i: (0, i, 0)),
            pl.BlockSpec((2, BLK, HALF), lambda i: (0, i, 0)),
            pl.BlockSpec((HD, HID), lambda i: (0, 0)),
            pl.BlockSpec((1, HID), lambda i: (0, 0)),
            pl.BlockSpec((1, HID), lambda i: (0, 0)),
        ],
        out_specs=pl.BlockSpec((1, 2), lambda i: (0, 0)),
        out_shape=jax.ShapeDtypeStruct((1, 2), jnp.float32),
    )(h1h, h2h, W1, b1r, w2r)


def _s3b_body(acc_ref, h1_ref, h2_ref, out_ref):
    w0 = acc_ref[0, 0] / N
    w1 = acc_ref[0, 1] / N
    m = jnp.maximum(w0, w1)
    e0 = jnp.exp(w0 - m)
    e1 = jnp.exp(w1 - m)
    bb0 = e0 / (e0 + e1)
    bb1 = e1 / (e0 + e1)
    left = bb0 * h1_ref[0] + bb1 * h2_ref[0]
    right = bb0 * h1_ref[1] + bb1 * h2_ref[1]
    out_ref[...] = jnp.concatenate([left, right], axis=1)


def _stage3b(acc, h1h, h2h):
    return pl.pallas_call(
        _s3b_body,
        grid=(NBLK,),
        in_specs=[
            pl.BlockSpec((1, 2), lambda i: (0, 0)),
            pl.BlockSpec((2, BLK, HALF), lambda i: (0, i, 0)),
            pl.BlockSpec((2, BLK, HALF), lambda i: (0, i, 0)),
        ],
        out_specs=pl.BlockSpec((BLK, HD), lambda i: (i, 0)),
        out_shape=jax.ShapeDtypeStruct((N, HD), jnp.float32),
    )(acc, h1h, h2h)


def kernel(x, edge_index1, edge_index2, W_gat, attn_l, attn_r, b_gat,
           W1, b1, W2):
    feat_h, elt, ert, _mraw, mx = _stage1(x, W_gat, attn_l, attn_r)
    feat2 = feat_h.reshape(2 * N, HALF)
    b2 = b_gat.reshape(2, HALF)
    zrows = jnp.zeros((NCH, HALF), jnp.float32)
    elf = elt.reshape(HEADS * N)
    erf = ert.reshape(HEADS * N)
    eis = jnp.stack([edge_index1, edge_index2]).reshape(4 * E)
    hs = _stage2(feat2, elf, erf, mx, b2, zrows, eis)
    h1h, h2h = hs[0], hs[1]
    acc = _stage3a(h1h, h2h, W1, b1.reshape(1, HID), W2.reshape(1, HID))
    return _stage3b(acc, h1h, h2h)


# speedup vs baseline: 1.0498x; 1.0020x over previous
import jax
import jax.numpy as jnp
from jax import lax
from jax.experimental import pallas as pl
from jax.experimental.pallas import tpu as pltpu
from jax.experimental.pallas import tpu_sc as plsc

N = 10000
D_IN = 256
HEADS = 4
D_OUT = 64
HID = 128
E = 160000
HD = HEADS * D_OUT
HALF = HD // 2

BLK = 400
NBLK = N // BLK
NTILE = 16
EPT = E // NTILE
ECH = 80
NCH_E = EPT // ECH
NCH = 80
NCHTOT = N // NCH
NSLOT = -(-NCHTOT // NTILE)


def _s1_body(x_ref, w_ref, al_ref, ar_ref, feat_ref, elt_ref, ert_ref, m_ref, mx_ref):
    i = pl.program_id(0)
    feat = jnp.dot(x_ref[...], w_ref[...], preferred_element_type=jnp.float32)
    els, ers = [], []
    for h in range(HEADS):
        fh = feat[:, h * D_OUT:(h + 1) * D_OUT]
        els.append((fh * al_ref[h, :][None, :]).sum(axis=1))
        ers.append((fh * ar_ref[h, :][None, :]).sum(axis=1))
    el = jnp.stack(els, axis=1)
    er = jnp.stack(ers, axis=1)
    feat_ref[0, :, :] = feat[:, :HALF]
    feat_ref[1, :, :] = feat[:, HALF:]
    elt_ref[...] = el
    ert_ref[...] = er
    pad = jnp.full((12,), -1e30, jnp.float32)
    mrow = jnp.stack([jnp.concatenate([jnp.max(el, axis=0), pad]),
                      jnp.concatenate([jnp.max(er, axis=0), pad])], axis=0)

    @pl.when(i == 0)
    def _():
        m_ref[...] = mrow

    @pl.when(i != 0)
    def _():
        m_ref[...] = jnp.maximum(m_ref[...], mrow)

    @pl.when(i == NBLK - 1)
    def _():
        m = m_ref[...]
        mv = jnp.maximum(m[0:1, :] + m[1:2, :], 0.0)
        li = lax.broadcasted_iota(jnp.int32, (2, 16), 1) & 1
        cc = lax.broadcasted_iota(jnp.int32, (2, 16), 0)
        hsel = 2 * cc + li
        mx = jnp.zeros((2, 16), jnp.float32)
        for h in range(HEADS):
            mx = jnp.where(hsel == h, mv[:, h:h + 1], mx)
        mx_ref[...] = mx


def _stage1(x, W_gat, attn_l, attn_r):
    return pl.pallas_call(
        _s1_body,
        grid=(NBLK,),
        in_specs=[
            pl.BlockSpec((BLK, D_IN), lambda i: (i, 0)),
            pl.BlockSpec((D_IN, HD), lambda i: (0, 0)),
            pl.BlockSpec((HEADS, D_OUT), lambda i: (0, 0)),
            pl.BlockSpec((HEADS, D_OUT), lambda i: (0, 0)),
        ],
        out_specs=[
            pl.BlockSpec((2, BLK, HALF), lambda i: (0, i, 0)),
            pl.BlockSpec((BLK, HEADS), lambda i: (i, 0)),
            pl.BlockSpec((BLK, HEADS), lambda i: (i, 0)),
            pl.BlockSpec((2, 16), lambda i: (0, 0)),
            pl.BlockSpec((2, 16), lambda i: (0, 0)),
        ],
        out_shape=[
            jax.ShapeDtypeStruct((2, N, HALF), jnp.float32),
            jax.ShapeDtypeStruct((N, HEADS), jnp.float32),
            jax.ShapeDtypeStruct((N, HEADS), jnp.float32),
            jax.ShapeDtypeStruct((2, 16), jnp.float32),
            jax.ShapeDtypeStruct((2, 16), jnp.float32),
        ],
    )(x, W_gat, attn_l, attn_r)


def _bcast_lane(v, lane):
    idx = jnp.full((16, 1), lane, jnp.int32)
    return lax.gather(
        v, idx,
        lax.GatherDimensionNumbers(offset_dims=(), collapsed_slice_dims=(0,),
                                   start_index_map=(0,)),
        (1,), mode=lax.GatherScatterMode.PROMISE_IN_BOUNDS)


def _sc_body(feat2, elf, erf, mm, b2, zrows, eis,
             hso,
             gbufs, idss, ifas, ifbs, ifcs, ifds, iffs,
             g0s, g1s, g2s, g3s, lsrcs, ldsts,
             dbufA, dbufB, zbuf, bbuf, mtmp, sems, ssems, lsems,
             rst_sh, den0_sh, den1_sh):
    c = lax.axis_index("c")
    s = lax.axis_index("s")

    for q in range(ECH // 16):
        zbuf[pl.ds(q * 16, 16)] = jnp.zeros((16,), jnp.float32)

    pltpu.sync_copy(mm.at[c], mtmp)
    mvec = mtmp[...]
    m0 = mvec[0]
    m1 = mvec[1]
    pltpu.sync_copy(b2.at[c], bbuf)

    ebase = s * EPT
    h0 = 2 * c
    h1 = 2 * c + 1
    cn = c * N

    sets = tuple(
        (idss[i], ifas[i], ifbs[i], ifcs[i], ifds[i], iffs[i],
         g0s[i], g1s[i], g2s[i], g3s[i], gbufs[i], sems[i], ssems[i])
        for i in range(4))

    def _meta(m, _carry):
        sbase = pl.multiple_of((2 * m) * E, 8)
        dbase = pl.multiple_of((2 * m + 1) * E, 8)
        def _zero(k, _):
            ci = s + NTILE * k

            @pl.when(ci < NCHTOT)
            def _():
                n0 = pl.multiple_of(ci * NCH, NCH)
                pltpu.async_copy(zrows, rst_sh.at[pl.ds(n0, NCH)], sems[0])
                pltpu.async_copy(zbuf, den0_sh.at[pl.ds(n0, NCH)], sems[1])
                pltpu.async_copy(zbuf, den1_sh.at[pl.ds(n0, NCH)], sems[1])
            return 0
        lax.fori_loop(0, NSLOT, _zero, 0)

        def _zdrain(k, _):
            ci = s + NTILE * k

            @pl.when(ci < NCHTOT)
            def _():
                n0 = pl.multiple_of(ci * NCH, NCH)
                pltpu.make_async_copy(
                    zrows, rst_sh.at[pl.ds(n0, NCH)], sems[0]).wait()
                pltpu.make_async_copy(
                    zbuf, den0_sh.at[pl.ds(n0, NCH)], sems[1]).wait()
                pltpu.make_async_copy(
                    zbuf, den1_sh.at[pl.ds(n0, NCH)], sems[1]).wait()
            return 0
        lax.fori_loop(0, NSLOT, _zdrain, 0)
        plsc.subcore_barrier()

        def _lfire(ch, par):
            e0 = ebase + ch * ECH
            pltpu.async_copy(eis.at[pl.ds(sbase + e0, ECH)], lsrcs[par],
                             lsems[par])
            pltpu.async_copy(eis.at[pl.ds(dbase + e0, ECH)], ldsts[par],
                             lsems[par])

        def _prep(bs, ch, par, drain):
            ids, fa, fb, fc, fd, ff, g0, g1, g2, g3, gb, sem, ssem = bs
            if drain:
                @pl.when(ch >= 4)
                def _():
                    pltpu.make_async_copy(gb, rst_sh.at[ids], ssem).wait()
                    pltpu.make_async_copy(g0, den0_sh.at[ids], ssem).wait()
                    pltpu.make_async_copy(g1, den1_sh.at[ids], ssem).wait()
            e0 = ebase + ch * ECH
            lsr = lsrcs[par]
            lds = ldsts[par]
            pltpu.make_async_copy(
                eis.at[pl.ds(sbase + e0, ECH)], lsr, lsems[par]).wait()
            pltpu.make_async_copy(
                eis.at[pl.ds(dbase + e0, ECH)], lds, lsems[par]).wait()
            def _pidx(q, _):
                sl = pl.ds(q * 16, 16)
                sv = lsr[sl]
                dv = lds[sl]
                sv4 = sv * 4
                dv4 = dv * 4
                fa[sl] = sv4 + h0
                fb[sl] = sv4 + h1
                fc[sl] = dv4 + h0
                fd[sl] = dv4 + h1
                ff[sl] = sv + cn
                ids[sl] = dv
                return 0
            lax.fori_loop(0, ECH // 16, _pidx, 0)

            @pl.when(ch + 2 < NCH_E)
            def _():
                _lfire(ch + 2, par)
            pltpu.async_copy(elf.at[fa], g0, sem)
            pltpu.async_copy(elf.at[fb], g1, sem)
            pltpu.async_copy(erf.at[fc], g2, sem)
            pltpu.async_copy(erf.at[fd], g3, sem)
            pltpu.async_copy(feat2.at[ff], gb, sem)

        def _work(bs, ch):
            ids, fa, fb, fc, fd, ff, g0, g1, g2, g3, gb, sem, ssem = bs
            pltpu.make_async_copy(elf.at[fa], g0, sem).wait()
            pltpu.make_async_copy(elf.at[fb], g1, sem).wait()
            pltpu.make_async_copy(erf.at[fc], g2, sem).wait()
            pltpu.make_async_copy(erf.at[fd], g3, sem).wait()
            pltpu.make_async_copy(feat2.at[ff], gb, sem).wait()
            def _pee(q, _):
                sl = pl.ds(q * 16, 16)
                x0 = g0[sl] + g2[sl]
                x0 = jnp.maximum(x0, 0.2 * x0)
                v0 = jnp.exp(x0 - m0)
                x1 = g1[sl] + g3[sl]
                x1 = jnp.maximum(x1, 0.2 * x1)
                v1 = jnp.exp(x1 - m1)
                g0[sl] = v0
                g1[sl] = v1
                return 0
            lax.fori_loop(0, ECH // 16, _pee, 0)

            def _scale(g, _):
                ea = g0[pl.ds(g * 16, 16)]
                eb = g1[pl.ds(g * 16, 16)]
                for e16 in range(16):
                    e = g * 16 + e16
                    s0v = _bcast_lane(ea, e16)
                    s1v = _bcast_lane(eb, e16)
                    for j in range(8):
                        scv = s0v if j < 4 else s1v
                        gb[e, pl.ds(j * 16, 16)] = (
                            gb[e, pl.ds(j * 16, 16)] * scv)
                return 0
            lax.fori_loop(0, ECH // 16, _scale, 0)
            pltpu.async_copy(gb, rst_sh.at[ids], ssem, add=True)
            pltpu.async_copy(g0, den0_sh.at[ids], ssem, add=True)
            pltpu.async_copy(g1, den1_sh.at[ids], ssem, add=True)

        _lfire(0, 0)
        _lfire(1, 1)
        _prep(sets[0], 0, 0, False)

        def _pmain(p, _):
            for b in range(4):
                ch = 4 * p + b

                @pl.when(ch + 1 < NCH_E)
                def _(ch=ch, b=b):
                    _prep(sets[(b + 1) % 4], ch + 1, (b + 1) % 2, True)

                @pl.when(ch < NCH_E)
                def _(ch=ch, b=b):
                    _work(sets[b], ch)
            return 0
        lax.fori_loop(0, (NCH_E + 3) // 4, _pmain, 0)
        for i in range(4):
            ids, fa, fb, fc, fd, ff, g0, g1, g2, g3, gb, sem, ssem = \
                sets[i]
            pltpu.make_async_copy(gb, rst_sh.at[ids], ssem).wait()
            pltpu.make_async_copy(g0, den0_sh.at[ids], ssem).wait()
            pltpu.make_async_copy(g1, den1_sh.at[ids], ssem).wait()

        plsc.subcore_barrier()

        def _p3prep(k, b):
            ci = s + NTILE * k

            @pl.when(ci < NCHTOT)
            def _():
                n0 = pl.multiple_of(ci * NCH, NCH)
                gb = gbufs[b]
                if True:
                    @pl.when(k >= 2)
                    def _():
                        ci2 = s + NTILE * (k - 2)
                        n2 = pl.multiple_of(ci2 * NCH, NCH)
                        pltpu.make_async_copy(
                            gb, hso.at[m, c, pl.ds(n2, NCH), :],
                            ssems[b]).wait()
                pltpu.async_copy(rst_sh.at[pl.ds(n0, NCH)], gb, sems[b])
                pltpu.async_copy(den0_sh.at[pl.ds(n0, NCH)],
                                 dbufA[b], sems[b])
                pltpu.async_copy(den1_sh.at[pl.ds(n0, NCH)],
                                 dbufB[b], sems[b])

        def _p3work(k, b):
            ci = s + NTILE * k

            @pl.when(ci < NCHTOT)
            def _():
                n0 = pl.multiple_of(ci * NCH, NCH)
                gb = gbufs[b]
                pltpu.make_async_copy(
                    rst_sh.at[pl.ds(n0, NCH)], gb, sems[b]).wait()
                pltpu.make_async_copy(
                    den0_sh.at[pl.ds(n0, NCH)], dbufA[b], sems[b]).wait()
                pltpu.make_async_copy(
                    den1_sh.at[pl.ds(n0, NCH)], dbufB[b], sems[b]).wait()

                def _p3(g, _):
                    r0v = 1.0 / jnp.maximum(dbufA[b][pl.ds(g * 16, 16)],
                                            1e-9)
                    r1v = 1.0 / jnp.maximum(dbufB[b][pl.ds(g * 16, 16)],
                                            1e-9)
                    for n16 in range(16):
                        n = g * 16 + n16
                        r0 = _bcast_lane(r0v, n16)
                        r1 = _bcast_lane(r1v, n16)
                        for j in range(8):
                            r = r0 if j < 4 else r1
                            v = (gb[n, pl.ds(j * 16, 16)] * r
                                 + bbuf[pl.ds(j * 16, 16)])
                            v = jnp.where(v > 0, v,
                                          jnp.exp(jnp.minimum(v, 0.0)) - 1.0)
                            gb[n, pl.ds(j * 16, 16)] = v
                    return 0
                lax.fori_loop(0, NCH // 16, _p3, 0)
                pltpu.async_copy(gb, hso.at[m, c, pl.ds(n0, NCH), :], ssems[b])

        _p3prep(0, 0)

        def _p3outer(q, _):
            for b in range(2):
                k = 2 * q + b
                if True:
                    @pl.when(k + 1 < NSLOT)
                    def _(k=k, b=b):
                        _p3prep(k + 1, 1 - b)
                _p3work(k, b)
            return 0
        lax.fori_loop(0, NSLOT // 2, _p3outer, 0)
        for k, b in ((NSLOT - 2, 0), (NSLOT - 1, 1)):
            ci = s + NTILE * k

            @pl.when(ci < NCHTOT)
            def _(k=k, b=b, ci=ci):
                n0 = pl.multiple_of(ci * NCH, NCH)
                pltpu.make_async_copy(
                    gbufs[b], hso.at[m, c, pl.ds(n0, NCH), :], ssems[b]).wait()
        plsc.subcore_barrier()
        return 0

    lax.fori_loop(0, 2, _meta, 0)


def _stage2(feat2, elf, erf, mm, b2, zrows, eis):
    i32 = jnp.int32
    f32 = jnp.float32
    fn = pl.kernel(
        _sc_body,
        out_type=jax.ShapeDtypeStruct((2, 2, N, HALF), f32),
        mesh=plsc.VectorSubcoreMesh(core_axis_name="c", subcore_axis_name="s"),
        scratch_types=[
            [pltpu.VMEM((NCH, HALF), f32)] * 4,
            [pltpu.VMEM((ECH,), i32)] * 4,
            [pltpu.VMEM((ECH,), i32)] * 4,
            [pltpu.VMEM((ECH,), i32)] * 4,
            [pltpu.VMEM((ECH,), i32)] * 4,
            [pltpu.VMEM((ECH,), i32)] * 4,
            [pltpu.VMEM((ECH,), i32)] * 4,
            [pltpu.VMEM((ECH,), f32)] * 4,
            [pltpu.VMEM((ECH,), f32)] * 4,
            [pltpu.VMEM((ECH,), f32)] * 4,
            [pltpu.VMEM((ECH,), f32)] * 4,
            [pltpu.VMEM((ECH,), i32)] * 2,
            [pltpu.VMEM((ECH,), i32)] * 2,
            [pltpu.VMEM((NCH,), f32)] * 2,
            [pltpu.VMEM((NCH,), f32)] * 2,
            pltpu.VMEM((ECH,), f32),
            pltpu.VMEM((HALF,), f32),
            pltpu.VMEM((16,), f32),
            [pltpu.SemaphoreType.DMA] * 4,
            [pltpu.SemaphoreType.DMA] * 4,
            [pltpu.SemaphoreType.DMA] * 2,
            pltpu.VMEM_SHARED((N, HALF), f32),
            pltpu.VMEM_SHARED((N,), f32),
            pltpu.VMEM_SHARED((N,), f32),
        ],
    )
    return fn(feat2, elf, erf, mm, b2, zrows, eis)


def _s3_body(h1_ref, h2_ref, w1_ref, b1_ref, w2_ref, out_ref, acc_ref):
    i = pl.program_id(0)

    @pl.when(i < NBLK)
    def _():
        z1 = jnp.concatenate([h1_ref[0], h1_ref[1]], axis=1)
        z2 = jnp.concatenate([h2_ref[0], h2_ref[1]], axis=1)
        t1 = jnp.tanh(jnp.dot(z1, w1_ref[...],
                              preferred_element_type=jnp.float32)
                      + b1_ref[...])
        t2 = jnp.tanh(jnp.dot(z2, w1_ref[...],
                              preferred_element_type=jnp.float32)
                      + b1_ref[...])
        s1 = jnp.sum(t1 * w2_ref[...])
        s2 = jnp.sum(t2 * w2_ref[...])
        row = jnp.stack([s1, s2]).reshape(1, 2)

        @pl.when(i == 0)
        def _():
            acc_ref[...] = row

        @pl.when(i != 0)
        def _():
            acc_ref[...] = acc_ref[...] + row

    @pl.when(i >= NBLK)
    def _():
        w0 = acc_ref[0, 0] / N
        w1 = acc_ref[0, 1] / N
        m = jnp.maximum(w0, w1)
        e0 = jnp.exp(w0 - m)
        e1 = jnp.exp(w1 - m)
        bb0 = e0 / (e0 + e1)
        bb1 = e1 / (e0 + e1)
        left = bb0 * h1_ref[0] + bb1 * h2_ref[0]
        right = bb0 * h1_ref[1] + bb1 * h2_ref[1]
        out_ref[...] = jnp.concatenate([left, right], axis=1)


def _stage3(h1h, h2h, W1, b1r, w2r):
    def hmap(i):
        j = jnp.where(i < NBLK, i, i - NBLK)
        return (0, j, 0)

    return pl.pallas_call(
        _s3_body,
        grid=(2 * NBLK,),
        in_specs=[
            pl.BlockSpec((2, BLK, HALF), hmap),
            pl.BlockSpec((2, BLK, HALF), hmap),
            pl.BlockSpec((HD, HID), lambda i: (0, 0)),
            pl.BlockSpec((1, HID), lambda i: (0, 0)),
            pl.BlockSpec((1, HID), lambda i: (0, 0)),
        ],
        out_specs=pl.BlockSpec(
            (BLK, HD),
            lambda i: (jnp.where(i < NBLK, 0, i - NBLK), 0)),
        out_shape=jax.ShapeDtypeStruct((N, HD), jnp.float32),
        scratch_shapes=[pltpu.VMEM((1, 2), jnp.float32)],
    )(h1h, h2h, W1, b1r, w2r)


def kernel(x, edge_index1, edge_index2, W_gat, attn_l, attn_r, b_gat,
           W1, b1, W2):
    feat_h, elt, ert, _mraw, mx = _stage1(x, W_gat, attn_l, attn_r)
    feat2 = feat_h.reshape(2 * N, HALF)
    b2 = b_gat.reshape(2, HALF)
    zrows = jnp.zeros((NCH, HALF), jnp.float32)
    elf = elt.reshape(HEADS * N)
    erf = ert.reshape(HEADS * N)
    eis = jnp.stack([edge_index1, edge_index2]).reshape(4 * E)
    hs = _stage2(feat2, elf, erf, mx, b2, zrows, eis)
    h1h, h2h = hs[0], hs[1]
    return _stage3(h1h, h2h, W1, b1.reshape(1, HID), W2.reshape(1, HID))
